# baseline (speedup 1.0000x reference)
import functools

import jax
import jax.numpy as jnp
from jax import lax
from jax.experimental import pallas as pl
from jax.experimental.pallas import tpu as pltpu
from jax.experimental.pallas import tpu_sc as plsc

NS = 16
NSC = 2
CK = 32
GC = 8


def _tc_body(x_ref, w_ref, s_ref, b_ref, y_ref, z_ref):
    xb = x_ref[...]
    y_ref[...] = jnp.dot(xb, w_ref[...], preferred_element_type=jnp.float32)
    z_ref[...] = jnp.dot(xb, s_ref[...], preferred_element_type=jnp.float32) + b_ref[...]


def _tc_prep(x, weight, self_loops, bias):
    N, D = x.shape
    R = 1000
    grid = (N // R,)
    return pl.pallas_call(
        _tc_body,
        grid=grid,
        in_specs=[
            pl.BlockSpec((R, D), lambda i: (i, 0)),
            pl.BlockSpec((D, D), lambda i: (0, 0)),
            pl.BlockSpec((D, D), lambda i: (0, 0)),
            pl.BlockSpec((1, D), lambda i: (0, 0)),
        ],
        out_specs=[
            pl.BlockSpec((R, D), lambda i: (i, 0)),
            pl.BlockSpec((R, D), lambda i: (i, 0)),
        ],
        out_shape=[
            jax.ShapeDtypeStruct((N, D), jnp.float32),
            jax.ShapeDtypeStruct((N, D), jnp.float32),
        ],
    )(x, weight, self_loops, bias.reshape(1, D))


def _make_sc_kernel(N, D, GROUPS, GB0, GB1):
    HN = N // 2
    rows = -(-N // (NS * 8)) * 8
    last = N - (NS - 1) * rows
    yrows = -(-HN // (NS * 8)) * 8
    ylast = HN - (NS - 1) * yrows
    mesh = plsc.VectorSubcoreMesh(core_axis_name="c", subcore_axis_name="s")

    @functools.partial(
        pl.kernel,
        out_type=jax.ShapeDtypeStruct((NSC, N, D), jnp.float32),
        mesh=mesh,
        scratch_types=[
            pltpu.VMEM((2, 1, GC, 2 * CK), jnp.int32),
            pltpu.VMEM((2, CK, D), jnp.float32),
            pltpu.VMEM_SHARED((HN + 8, D), jnp.float32),
            pltpu.VMEM_SHARED((N + 8, D), jnp.float32),
            pltpu.SemaphoreType.DMA,
            pltpu.SemaphoreType.DMA,
            pltpu.SemaphoreType.DMA,
        ],
    )
    def sc_fn(y_hbm, z_hbm, zero_hbm, mrg_hbm, mrgb_hbm, out_hbm,
              mrg_v, gbuf, y_sh, acc_sh, gsem, ssem, rsem):
        c = lax.axis_index("c")
        s = lax.axis_index("s")
        base = s * GROUPS

        def drain_buf(sem):
            pltpu.make_async_copy(y_hbm.at[pl.ds(0, CK)], gbuf.at[0], sem).wait()

        def drain_ring(sem):
            pltpu.make_async_copy(mrg_hbm.at[pl.ds(0, 1)], mrg_v.at[0], sem).wait()

        @pl.when(s < NS - 1)
        def _():
            pltpu.sync_copy(y_hbm.at[pl.ds(c * HN + s * yrows, yrows)],
                            y_sh.at[pl.ds(s * yrows, yrows)])

        @pl.when(s == NS - 1)
        def _():
            pltpu.sync_copy(y_hbm.at[pl.ds(c * HN + (NS - 1) * yrows, ylast)],
                            y_sh.at[pl.ds((NS - 1) * yrows, ylast)])
            pltpu.sync_copy(zero_hbm.at[pl.ds(0, 8)], y_sh.at[pl.ds(HN, 8)])

        @pl.when(c == 0)
        def _():
            @pl.when(s < NS - 1)
            def _():
                pltpu.sync_copy(z_hbm.at[pl.ds(s * rows, rows)],
                                acc_sh.at[pl.ds(s * rows, rows)])

            @pl.when(s == NS - 1)
            def _():
                pltpu.sync_copy(z_hbm.at[pl.ds((NS - 1) * rows, last)],
                                acc_sh.at[pl.ds((NS - 1) * rows, last)])

        @pl.when((c == 1) & (s == 0))
        def _():
            pltpu.sync_copy(zero_hbm, acc_sh.at[pl.ds(0, 128)])

        plsc.subcore_barrier()

        @pl.when(c == 1)
        def _():
            nfull, rem = divmod(rows, 128)
            for t in range(nfull):
                @pl.when((s > 0) | (t > 0))
                def _():
                    pltpu.sync_copy(acc_sh.at[pl.ds(0, 128)],
                                    acc_sh.at[pl.ds(s * rows + t * 128, 128)])
            if rem:
                @pl.when(s < NS - 1)
                def _():
                    pltpu.sync_copy(acc_sh.at[pl.ds(0, rem)],
                                    acc_sh.at[pl.ds(s * rows + nfull * 128, rem)])
            lrem = last - (last // 128) * 128
            if lrem:
                @pl.when(s == NS - 1)
                def _():
                    pltpu.sync_copy(acc_sh.at[pl.ds(0, lrem)],
                                    acc_sh.at[pl.ds((NS - 1) * rows + (last // 128) * 128, lrem)])

        plsc.subcore_barrier()

        lo = c * HN

        def remap(p):
            for r in range(GC):
                for k in range(CK // 16):
                    v = mrg_v[p, 0, r, pl.ds(k * 16, 16)]
                    inr = (v >= lo) & (v < lo + HN)
                    mrg_v[p, 0, r, pl.ds(k * 16, 16)] = jnp.where(inr, v - lo, HN)

        def gidx(p, j):
            return mrg_v.at[p, 0, j, pl.ds(0, CK)]

        def sidx(p, j):
            return mrg_v.at[p, 0, j, pl.ds(CK, CK)]

        def run_phase(idx_hbm, pbase, ng, do_remap):
            def fire_g(p, j, b):
                src = y_sh if do_remap else y_hbm
                pltpu.async_copy(src.at[gidx(p, j)], gbuf.at[b], gsem)

            pltpu.async_copy(idx_hbm.at[pl.ds(pbase, 1)], mrg_v.at[0], rsem)
            drain_ring(rsem)
            if do_remap:
                remap(0)
            fire_g(0, 0, 0)

            def group(g, carry):
                p = lax.rem(g, 2)
                pn = lax.rem(g + 1, 2)

                @pl.when(g + 1 < ng)
                def _():
                    pltpu.async_copy(idx_hbm.at[pl.ds(pbase + g + 1, 1)],
                                     mrg_v.at[pn], rsem)

                for j in range(GC):
                    if j == 0:
                        @pl.when(g > 0)
                        def _():
                            drain_buf(ssem)
                    else:
                        drain_buf(ssem)
                    if j + 1 < GC:
                        fire_g(p, j + 1, (j + 1) % 2)
                    drain_buf(gsem)
                    pltpu.async_copy(gbuf.at[j % 2], acc_sh.at[sidx(p, j)],
                                     ssem, add=True)

                @pl.when(g + 1 < ng)
                def _():
                    drain_ring(rsem)
                    if do_remap:
                        remap(pn)
                    fire_g(pn, 0, 0)

                return carry

            lax.fori_loop(0, ng, group, 0)
            drain_buf(ssem)

        run_phase(mrg_hbm, base, GROUPS, True)
        base_b = jnp.where(c == 0, s * GB0, NS * GB0 + s * GB1)
        ng_b = jnp.where(c == 0, GB0, GB1)
        run_phase(mrgb_hbm, base_b, ng_b, False)
        plsc.subcore_barrier()

        @pl.when(s < NS - 1)
        def _():
            pltpu.sync_copy(acc_sh.at[pl.ds(s * rows, rows)],
                            out_hbm.at[c, pl.ds(s * rows, rows)])

        @pl.when(s == NS - 1)
        def _():
            pltpu.sync_copy(acc_sh.at[pl.ds((NS - 1) * rows, last)],
                            out_hbm.at[c, pl.ds((NS - 1) * rows, last)])

    return sc_fn


def kernel(x, edge_index, weight, self_loops, bias):
    N, D = x.shape
    E = edge_index.shape[0]
    unit = GC * CK
    GROUPS = max(1, round(0.46 * E / (NS * unit)))
    EA = NS * GROUPS * unit
    B = E - EA
    gbt = -(-B // (NS * unit))
    GB1 = max(1, round(gbt / 3.0))
    GB0 = gbt - GB1
    padb = NS * gbt * unit - B

    y, z = _tc_prep(x, weight, self_loops, bias)

    src = edge_index[:, 0]
    dst = edge_index[:, 1]
    mrg = jnp.concatenate(
        [dst[:EA].reshape(NS * GROUPS, GC, CK),
         src[:EA].reshape(NS * GROUPS, GC, CK)], axis=2)
    dstb = jnp.concatenate([dst[EA:], jnp.zeros((padb,), jnp.int32)])
    srcb = jnp.concatenate([src[EA:], jnp.full((padb,), N, jnp.int32)])
    mrgb = jnp.concatenate(
        [dstb.reshape(NS * gbt, GC, CK), srcb.reshape(NS * gbt, GC, CK)], axis=2)
    zero_blk = jnp.zeros((128, D), jnp.float32)

    out_sc = _make_sc_kernel(N, D, GROUPS, GB0, GB1)(y, z, zero_blk, mrg, mrgb)
    return out_sc[0] + out_sc[1]


# speedup vs baseline: 1.1726x; 1.1726x over previous
import functools

import jax
import jax.numpy as jnp
from jax import lax
from jax.experimental import pallas as pl
from jax.experimental.pallas import tpu as pltpu
from jax.experimental.pallas import tpu_sc as plsc

NS = 16
NSC = 2
CK = 32
GC = 4


def _tc_body(x_ref, w_ref, s_ref, b_ref, y_ref, z_ref):
    xb = x_ref[...]
    y_ref[...] = jnp.dot(xb, w_ref[...], preferred_element_type=jnp.float32)
    z_ref[...] = jnp.dot(xb, s_ref[...], preferred_element_type=jnp.float32) + b_ref[...]


def _tc_prep(x, weight, self_loops, bias):
    N, D = x.shape
    R = 1000
    grid = (N // R,)
    return pl.pallas_call(
        _tc_body,
        grid=grid,
        in_specs=[
            pl.BlockSpec((R, D), lambda i: (i, 0)),
            pl.BlockSpec((D, D), lambda i: (0, 0)),
            pl.BlockSpec((D, D), lambda i: (0, 0)),
            pl.BlockSpec((1, D), lambda i: (0, 0)),
        ],
        out_specs=[
            pl.BlockSpec((R, D), lambda i: (i, 0)),
            pl.BlockSpec((R, D), lambda i: (i, 0)),
        ],
        out_shape=[
            jax.ShapeDtypeStruct((N, D), jnp.float32),
            jax.ShapeDtypeStruct((N, D), jnp.float32),
        ],
    )(x, weight, self_loops, bias.reshape(1, D))


def _make_sc_kernel(N, D, GROUPS):
    HN = N // 2
    rows = -(-N // (NS * 8)) * 8
    last = N - (NS - 1) * rows
    yrows = -(-HN // (NS * 8)) * 8
    ylast = HN - (NS - 1) * yrows
    mesh = plsc.VectorSubcoreMesh(core_axis_name="c", subcore_axis_name="s")

    @functools.partial(
        pl.kernel,
        out_type=jax.ShapeDtypeStruct((NSC, N, D), jnp.float32),
        mesh=mesh,
        scratch_types=[
            pltpu.VMEM((2, 1, GC, CK), jnp.int32),
            pltpu.VMEM((2, 1, GC, CK), jnp.int32),
            pltpu.VMEM((2, CK, D), jnp.float32),
            pltpu.VMEM_SHARED((HN + 8, D), jnp.float32),
            pltpu.VMEM_SHARED((N, D), jnp.float32),
            pltpu.SemaphoreType.DMA,
            pltpu.SemaphoreType.DMA,
            pltpu.SemaphoreType.DMA,
        ],
    )
    def sc_fn(y_hbm, z_hbm, zero_hbm, src_hbm, dst_hbm, out_hbm,
              src_v, dst_v, gbuf, y_sh, acc_sh, gsem, ssem, rsem):
        c = lax.axis_index("c")
        s = lax.axis_index("s")
        base = s * GROUPS

        def drain_buf(sem):
            pltpu.make_async_copy(y_hbm.at[pl.ds(0, CK)], gbuf.at[0], sem).wait()

        def drain_ring(sem):
            pltpu.make_async_copy(src_hbm.at[pl.ds(0, 1)], src_v.at[0], sem).wait()

        @pl.when(s < NS - 1)
        def _():
            pltpu.sync_copy(y_hbm.at[pl.ds(c * HN + s * yrows, yrows)],
                            y_sh.at[pl.ds(s * yrows, yrows)])

        @pl.when(s == NS - 1)
        def _():
            pltpu.sync_copy(y_hbm.at[pl.ds(c * HN + (NS - 1) * yrows, ylast)],
                            y_sh.at[pl.ds((NS - 1) * yrows, ylast)])
            pltpu.sync_copy(zero_hbm.at[pl.ds(0, 8)], y_sh.at[pl.ds(HN, 8)])

        @pl.when(c == 0)
        def _():
            @pl.when(s < NS - 1)
            def _():
                pltpu.sync_copy(z_hbm.at[pl.ds(s * rows, rows)],
                                acc_sh.at[pl.ds(s * rows, rows)])

            @pl.when(s == NS - 1)
            def _():
                pltpu.sync_copy(z_hbm.at[pl.ds((NS - 1) * rows, last)],
                                acc_sh.at[pl.ds((NS - 1) * rows, last)])

        @pl.when((c == 1) & (s == 0))
        def _():
            pltpu.sync_copy(zero_hbm, acc_sh.at[pl.ds(0, 128)])

        plsc.subcore_barrier()

        @pl.when(c == 1)
        def _():
            nfull, rem = divmod(rows, 128)
            for t in range(nfull):
                @pl.when((s > 0) | (t > 0))
                def _():
                    pltpu.sync_copy(acc_sh.at[pl.ds(0, 128)],
                                    acc_sh.at[pl.ds(s * rows + t * 128, 128)])
            if rem:
                @pl.when(s < NS - 1)
                def _():
                    pltpu.sync_copy(acc_sh.at[pl.ds(0, rem)],
                                    acc_sh.at[pl.ds(s * rows + nfull * 128, rem)])
            lrem = last - (last // 128) * 128
            if lrem:
                @pl.when(s == NS - 1)
                def _():
                    pltpu.sync_copy(acc_sh.at[pl.ds(0, lrem)],
                                    acc_sh.at[pl.ds((NS - 1) * rows + (last // 128) * 128, lrem)])

        pltpu.async_copy(src_hbm.at[pl.ds(base, 1)], src_v.at[0], rsem)
        pltpu.async_copy(dst_hbm.at[pl.ds(base, 1)], dst_v.at[0], rsem)
        plsc.subcore_barrier()

        lo = c * HN

        def remap(p):
            for r in range(GC):
                for k in range(CK // 16):
                    v = dst_v[p, 0, r, pl.ds(k * 16, 16)]
                    inr = (v >= lo) & (v < lo + HN)
                    dst_v[p, 0, r, pl.ds(k * 16, 16)] = jnp.where(inr, v - lo, HN)

        drain_ring(rsem)
        drain_ring(rsem)
        remap(0)
        pltpu.async_copy(y_sh.at[dst_v.at[0, 0, 0]], gbuf.at[0], gsem)

        def group(g, carry):
            p = lax.rem(g, 2)
            pn = lax.rem(g + 1, 2)

            @pl.when(g + 1 < GROUPS)
            def _():
                pltpu.async_copy(src_hbm.at[pl.ds(base + g + 1, 1)], src_v.at[pn], rsem)
                pltpu.async_copy(dst_hbm.at[pl.ds(base + g + 1, 1)], dst_v.at[pn], rsem)

            for j in range(GC):
                if j == 0:
                    @pl.when(g > 0)
                    def _():
                        drain_buf(ssem)
                else:
                    drain_buf(ssem)
                if j + 1 < GC:
                    pltpu.async_copy(y_sh.at[dst_v.at[p, 0, j + 1]],
                                     gbuf.at[(j + 1) % 2], gsem)
                drain_buf(gsem)
                pltpu.async_copy(gbuf.at[j % 2], acc_sh.at[src_v.at[p, 0, j]],
                                 ssem, add=True)

            @pl.when(g + 1 < GROUPS)
            def _():
                drain_ring(rsem)
                drain_ring(rsem)
                remap(pn)
                pltpu.async_copy(y_sh.at[dst_v.at[pn, 0, 0]], gbuf.at[0], gsem)

            return carry

        lax.fori_loop(0, GROUPS, group, 0)
        drain_buf(ssem)
        plsc.subcore_barrier()

        @pl.when(s < NS - 1)
        def _():
            pltpu.sync_copy(acc_sh.at[pl.ds(s * rows, rows)],
                            out_hbm.at[c, pl.ds(s * rows, rows)])

        @pl.when(s == NS - 1)
        def _():
            pltpu.sync_copy(acc_sh.at[pl.ds((NS - 1) * rows, last)],
                            out_hbm.at[c, pl.ds((NS - 1) * rows, last)])

    return sc_fn


def kernel(x, edge_index, weight, self_loops, bias):
    N, D = x.shape
    E = edge_index.shape[0]
    GROUPS = -(-E // (NS * GC * CK))
    EP = NS * GROUPS * GC * CK
    pad = EP - E

    y, z = _tc_prep(x, weight, self_loops, bias)

    src = edge_index[:, 0]
    dst = edge_index[:, 1]
    srcp = jnp.concatenate([src, jnp.zeros((pad,), jnp.int32)]).reshape(NS * GROUPS, GC, CK)
    dstp = jnp.concatenate([dst, jnp.full((pad,), N, jnp.int32)]).reshape(NS * GROUPS, GC, CK)
    zero_blk = jnp.zeros((128, D), jnp.float32)

    out_sc = _make_sc_kernel(N, D, GROUPS)(y, z, zero_blk, srcp, dstp)
    return out_sc[0] + out_sc[1]
